# Initial kernel scaffold; baseline (speedup 1.0000x reference)
import jax
import jax.numpy as jnp
from jax.experimental import pallas as pl


def kernel(ze, book, log_param_q, is_train):
    raise NotImplementedError("write your pallas kernel here")


# speedup vs baseline: 2.6325x; 2.6325x over previous
import jax
import jax.numpy as jnp
from jax.experimental import pallas as pl

_BOOK_SIZE = 1024
_LATENT = 256
_BLK = 512


def _vq_block_kernel(pq_ref, x_ref, book_ref, prob_ref, logp_ref, zq_ref):
    pq = pq_ref[0, 0]
    x = x_ref[...]
    bk = book_ref[...]
    g = jax.lax.dot_general(x, bk, (((1,), (1,)), ((), ())),
                            preferred_element_type=jnp.float32)
    xx = jnp.sum(x * x, axis=1, keepdims=True)
    bb = jnp.sum(bk * bk, axis=1)[None, :]
    dist = (xx + bb) - 2.0 * g
    logits = -dist * pq
    m = jnp.max(logits, axis=1, keepdims=True)
    e = jnp.exp(logits - m)
    s = jnp.sum(e, axis=1, keepdims=True)
    prob_ref[...] = e / s
    logp_ref[...] = (logits - m) - jnp.log(s)
    iota = jax.lax.broadcasted_iota(jnp.int32, (x.shape[0], _BOOK_SIZE), 1)
    masked = jnp.where(logits == m, iota, _BOOK_SIZE)
    idx = jnp.min(masked, axis=1, keepdims=True)
    onehot = (iota == idx).astype(jnp.float32)
    zq_ref[...] = jax.lax.dot_general(onehot, bk, (((1,), (0,)), ((), ())),
                                      preferred_element_type=jnp.float32)


def kernel(ze, book, log_param_q, is_train):
    b, n_pts, latent_ndim = ze.shape
    param_q = 1.0 + jnp.exp(log_param_q)
    precision_q = 0.5 / jnp.maximum(param_q, 1e-10)
    x = jnp.transpose(ze, (0, 2, 1)).reshape(-1, latent_ndim)
    rows = x.shape[0]
    pq_arr = jnp.reshape(precision_q, (1, 1))
    grid = (rows // _BLK,)
    prob, logp, zq = pl.pallas_call(
        _vq_block_kernel,
        grid=grid,
        in_specs=[
            pl.BlockSpec((1, 1), lambda i: (0, 0)),
            pl.BlockSpec((_BLK, _LATENT), lambda i: (i, 0)),
            pl.BlockSpec((_BOOK_SIZE, _LATENT), lambda i: (0, 0)),
        ],
        out_specs=[
            pl.BlockSpec((_BLK, _BOOK_SIZE), lambda i: (i, 0)),
            pl.BlockSpec((_BLK, _BOOK_SIZE), lambda i: (i, 0)),
            pl.BlockSpec((_BLK, _LATENT), lambda i: (i, 0)),
        ],
        out_shape=[
            jax.ShapeDtypeStruct((rows, _BOOK_SIZE), jnp.float32),
            jax.ShapeDtypeStruct((rows, _BOOK_SIZE), jnp.float32),
            jax.ShapeDtypeStruct((rows, latent_ndim), jnp.float32),
        ],
    )(pq_arr, x, book)
    zq = jnp.transpose(zq.reshape(b, latent_ndim, n_pts), (0, 2, 1))
    prob = prob.reshape(b, n_pts, _BOOK_SIZE)
    logp = logp.reshape(b, n_pts, _BOOK_SIZE)
    return (zq, precision_q, prob, logp)
